# baseline (speedup 1.0000x reference)
import jax
import jax.numpy as jnp
from jax.experimental import pallas as pl

HEADS = 8
DIM_HEAD = 32
DIM = 768
INNER = HEADS * DIM_HEAD
EPS = 1e-5


def _ln_qkv_kernel(x_ref, g_ref, bln_ref, wqkv_ref, q_ref, k_ref, v_ref):
    x = x_ref[0]
    ones_row = jnp.ones((8, DIM), dtype=jnp.float32)
    sums = jnp.dot(ones_row, x, preferred_element_type=jnp.float32)[:1]
    sqs = jnp.dot(ones_row, x * x, preferred_element_type=jnp.float32)[:1]
    mean = sums * (1.0 / DIM)
    var = sqs * (1.0 / DIM) - mean * mean
    xn = (x - mean) * jax.lax.rsqrt(var + EPS)
    xn = xn * g_ref[...].reshape(DIM, 1) + bln_ref[...].reshape(DIM, 1)
    qkv = jnp.dot(wqkv_ref[...], xn.astype(jnp.bfloat16),
                  preferred_element_type=jnp.float32)
    q_ref[0] = qkv[:INNER].astype(jnp.bfloat16)
    k_ref[0] = qkv[INNER:2 * INNER].astype(jnp.bfloat16)
    v_ref[0] = qkv[2 * INNER:].astype(jnp.bfloat16)


def _attn_proj_kernel(q_ref, k_ref, v_ref, wout_ref, bout_ref, o_ref):
    ones = jnp.ones((32, 8), dtype=jnp.float32)
    ys = []
    for hh in range(HEADS):
        q = q_ref[0, hh].astype(jnp.float32)
        k = k_ref[0, hh].astype(jnp.float32)
        v = v_ref[0, hh]
        sq = jnp.dot(q * q, ones, preferred_element_type=jnp.float32)[:, :1]
        sk = jnp.dot(k * k, ones, preferred_element_type=jnp.float32)[:, :1]
        qn = (q * jax.lax.rsqrt(jnp.maximum(sq, 1e-24))).astype(jnp.bfloat16)
        kn = (k * jax.lax.rsqrt(jnp.maximum(sk, 1e-24))).astype(jnp.bfloat16)
        sim = jnp.dot(qn, kn.T, preferred_element_type=jnp.float32)
        e = jnp.exp(sim).astype(jnp.bfloat16)
        vv = jnp.concatenate(
            [v, jnp.ones((v.shape[0], 8), dtype=jnp.bfloat16)], axis=1)
        os_ = jnp.dot(e, vv, preferred_element_type=jnp.float32)
        o = os_[:, :DIM_HEAD] / os_[:, DIM_HEAD:DIM_HEAD + 1]
        y = jnp.transpose(o.reshape(DIM_HEAD, 32, 32), (1, 2, 0))
        ys.append(y.reshape(32, DIM_HEAD * 32))
    yb = jnp.concatenate(ys, axis=0).astype(jnp.bfloat16)
    o_ref[0] = (jnp.dot(wout_ref[...], yb, preferred_element_type=jnp.float32)
                + bout_ref[...].reshape(DIM, 1))


def kernel(x, g, b_ln, W_qkv, W_out, b_out):
    b, c, h, w = x.shape
    hw = h * w
    xf = x.reshape(b, c, hw)
    gv = g.reshape(c)
    bv = b_ln.reshape(c)
    wqkv_b = W_qkv.astype(jnp.bfloat16)
    wout_b = W_out.astype(jnp.bfloat16)

    q, k, v = pl.pallas_call(
        _ln_qkv_kernel,
        grid=(b,),
        in_specs=[
            pl.BlockSpec((1, c, hw), lambda i: (i, 0, 0)),
            pl.BlockSpec((c,), lambda i: (0,)),
            pl.BlockSpec((c,), lambda i: (0,)),
            pl.BlockSpec((3 * INNER, c), lambda i: (0, 0)),
        ],
        out_specs=[
            pl.BlockSpec((1, INNER, hw), lambda i: (i, 0, 0)),
            pl.BlockSpec((1, INNER, hw), lambda i: (i, 0, 0)),
            pl.BlockSpec((1, INNER, hw), lambda i: (i, 0, 0)),
        ],
        out_shape=[
            jax.ShapeDtypeStruct((b, INNER, hw), jnp.bfloat16),
            jax.ShapeDtypeStruct((b, INNER, hw), jnp.bfloat16),
            jax.ShapeDtypeStruct((b, INNER, hw), jnp.bfloat16),
        ],
    )(xf, gv, bv, wqkv_b)

    n_tok = DIM_HEAD * h
    q = q.reshape(b, HEADS, n_tok, w)
    k = k.reshape(b, HEADS, n_tok, w)
    v = v.reshape(b, HEADS, n_tok, w)

    out = pl.pallas_call(
        _attn_proj_kernel,
        grid=(b,),
        in_specs=[
            pl.BlockSpec((1, HEADS, n_tok, w), lambda i: (i, 0, 0, 0)),
            pl.BlockSpec((1, HEADS, n_tok, w), lambda i: (i, 0, 0, 0)),
            pl.BlockSpec((1, HEADS, n_tok, w), lambda i: (i, 0, 0, 0)),
            pl.BlockSpec((DIM, INNER), lambda i: (0, 0)),
            pl.BlockSpec((DIM,), lambda i: (0,)),
        ],
        out_specs=pl.BlockSpec((1, DIM, hw), lambda i: (i, 0, 0)),
        out_shape=jax.ShapeDtypeStruct((b, DIM, hw), jnp.float32),
    )(q, k, v, wout_b, b_out)

    return out.reshape(b, DIM, h, w)


# speedup vs baseline: 1.0367x; 1.0367x over previous
import jax
import jax.numpy as jnp
from jax.experimental import pallas as pl

HEADS = 8
DIM_HEAD = 32
DIM = 768
INNER = HEADS * DIM_HEAD
EPS = 1e-5


def _ln_qkv_kernel(x_ref, g_ref, bln_ref, wqkv_ref, q_ref, k_ref, v_ref):
    x = x_ref[0]
    mean = jnp.mean(x, axis=0, keepdims=True)
    var = jnp.mean((x - mean) ** 2, axis=0, keepdims=True)
    xn = (x - mean) * jax.lax.rsqrt(var + EPS)
    xn = xn * g_ref[...].reshape(DIM, 1) + bln_ref[...].reshape(DIM, 1)
    qkv = jnp.dot(wqkv_ref[...], xn.astype(jnp.bfloat16),
                  preferred_element_type=jnp.float32)
    q_ref[0] = qkv[:INNER].astype(jnp.bfloat16)
    k_ref[0] = qkv[INNER:2 * INNER].astype(jnp.bfloat16)
    v_ref[0] = qkv[2 * INNER:].astype(jnp.bfloat16)


def _attn_proj_kernel(q_ref, k_ref, v_ref, wout_ref, bout_ref, o_ref):
    ones = jnp.ones((32, 8), dtype=jnp.float32)
    ys = []
    for hh in range(HEADS):
        q = q_ref[0, hh].astype(jnp.float32)
        k = k_ref[0, hh].astype(jnp.float32)
        v = v_ref[0, hh]
        sq = jnp.dot(q * q, ones, preferred_element_type=jnp.float32)[:, :1]
        sk = jnp.dot(k * k, ones, preferred_element_type=jnp.float32)[:, :1]
        qn = (q * jax.lax.rsqrt(jnp.maximum(sq, 1e-24))).astype(jnp.bfloat16)
        kn = (k * jax.lax.rsqrt(jnp.maximum(sk, 1e-24))).astype(jnp.bfloat16)
        sim = jnp.dot(qn, kn.T, preferred_element_type=jnp.float32)
        e = jnp.exp(sim).astype(jnp.bfloat16)
        vv = jnp.concatenate(
            [v, jnp.ones((v.shape[0], 8), dtype=jnp.bfloat16)], axis=1)
        os_ = jnp.dot(e, vv, preferred_element_type=jnp.float32)
        o = os_[:, :DIM_HEAD] / os_[:, DIM_HEAD:DIM_HEAD + 1]
        ob = o.astype(jnp.bfloat16)
        y = jnp.transpose(ob.reshape(DIM_HEAD, 32, 32), (1, 2, 0))
        ys.append(y.reshape(32, DIM_HEAD * 32))
    yb = jnp.concatenate(ys, axis=0)
    o_ref[0] = (jnp.dot(wout_ref[...], yb, preferred_element_type=jnp.float32)
                + bout_ref[...].reshape(DIM, 1))


def kernel(x, g, b_ln, W_qkv, W_out, b_out):
    b, c, h, w = x.shape
    hw = h * w
    xf = x.reshape(b, c, hw)
    gv = g.reshape(c)
    bv = b_ln.reshape(c)
    wqkv_b = W_qkv.astype(jnp.bfloat16)
    wout_b = W_out.astype(jnp.bfloat16)

    q, k, v = pl.pallas_call(
        _ln_qkv_kernel,
        grid=(b,),
        in_specs=[
            pl.BlockSpec((1, c, hw), lambda i: (i, 0, 0)),
            pl.BlockSpec((c,), lambda i: (0,)),
            pl.BlockSpec((c,), lambda i: (0,)),
            pl.BlockSpec((3 * INNER, c), lambda i: (0, 0)),
        ],
        out_specs=[
            pl.BlockSpec((1, INNER, hw), lambda i: (i, 0, 0)),
            pl.BlockSpec((1, INNER, hw), lambda i: (i, 0, 0)),
            pl.BlockSpec((1, INNER, hw), lambda i: (i, 0, 0)),
        ],
        out_shape=[
            jax.ShapeDtypeStruct((b, INNER, hw), jnp.bfloat16),
            jax.ShapeDtypeStruct((b, INNER, hw), jnp.bfloat16),
            jax.ShapeDtypeStruct((b, INNER, hw), jnp.bfloat16),
        ],
    )(xf, gv, bv, wqkv_b)

    n_tok = DIM_HEAD * h
    q = q.reshape(b, HEADS, n_tok, w)
    k = k.reshape(b, HEADS, n_tok, w)
    v = v.reshape(b, HEADS, n_tok, w)

    out = pl.pallas_call(
        _attn_proj_kernel,
        grid=(b,),
        in_specs=[
            pl.BlockSpec((1, HEADS, n_tok, w), lambda i: (i, 0, 0, 0)),
            pl.BlockSpec((1, HEADS, n_tok, w), lambda i: (i, 0, 0, 0)),
            pl.BlockSpec((1, HEADS, n_tok, w), lambda i: (i, 0, 0, 0)),
            pl.BlockSpec((DIM, INNER), lambda i: (0, 0)),
            pl.BlockSpec((DIM,), lambda i: (0,)),
        ],
        out_specs=pl.BlockSpec((1, DIM, hw), lambda i: (i, 0, 0)),
        out_shape=jax.ShapeDtypeStruct((b, DIM, hw), jnp.float32),
    )(q, k, v, wout_b, b_out)

    return out.reshape(b, DIM, h, w)
